# baseline (speedup 1.0000x reference)
import jax
import jax.numpy as jnp
from jax import lax
from jax.experimental import pallas as pl
from jax.experimental.pallas import tpu as pltpu
from jax.experimental.pallas import tpu_sc as plsc

N_VOCAB = 100000
N_EMBED = 128
N_TOKEN = 200
BATCH = 4096

NC = 2
NS = 16
NW = NC * NS

TOTAL = BATCH * N_TOKEN
PER_W = TOTAL // NW
CHUNK = 80
NCHUNK = PER_W // CHUNK
NBUF = 5
POS_EXT = 240

OFFS = tuple((CHUNK * c) % N_TOKEN for c in range(NBUF))


def _emb_kernel(token_hbm, table_hbm, pos_hbm, out_hbm,
                idx_v, buf0, buf1, buf2, buf3, buf4,
                fsem0, fsem1, fsem2, fsem3, fsem4,
                gsem0, gsem1, gsem2, gsem3, gsem4,
                wsem0, wsem1, wsem2, wsem3, wsem4):
    wid = lax.axis_index("s") * NC + lax.axis_index("c")
    base = wid * PER_W

    pltpu.sync_copy(token_hbm.at[wid], idx_v)

    bufs = (buf0, buf1, buf2, buf3, buf4)
    fsems = (fsem0, fsem1, fsem2, fsem3, fsem4)
    gsems = (gsem0, gsem1, gsem2, gsem3, gsem4)
    wsems = (wsem0, wsem1, wsem2, wsem3, wsem4)

    def fill(c, b):
        pltpu.async_copy(pos_hbm.at[pl.ds(OFFS[c % NBUF], CHUNK)],
                         bufs[b], fsems[b])

    for c in range(NBUF):
        fill(c, c)
    pltpu.make_async_copy(pos_hbm.at[pl.ds(0, CHUNK)], bufs[0], fsems[0]).wait()
    pltpu.async_copy(table_hbm.at[idx_v.at[0]], bufs[0], gsems[0], add=True)

    def step(m, _):
        for b in range(NBUF):
            h = NBUF * m + b

            pltpu.make_async_copy(table_hbm.at[idx_v.at[0]], bufs[b],
                                  gsems[b]).wait()
            pltpu.async_copy(bufs[b],
                             out_hbm.at[pl.ds(base + h * CHUNK, CHUNK)],
                             wsems[b])

            bn = (b + 3) % NBUF

            @pl.when(jnp.logical_and(h >= 2, h + 3 < NCHUNK))
            def _refill(bn=bn, b=b):
                pltpu.make_async_copy(bufs[bn], out_hbm.at[pl.ds(0, CHUNK)],
                                      wsems[bn]).wait()
                fill(b + 3, bn)

            bg = (b + 1) % NBUF

            @pl.when(h + 1 < NCHUNK)
            def _gather(bg=bg, h=h):
                pltpu.make_async_copy(pos_hbm.at[pl.ds(0, CHUNK)], bufs[bg],
                                      fsems[bg]).wait()
                pltpu.async_copy(table_hbm.at[idx_v.at[h + 1]], bufs[bg],
                                 gsems[bg], add=True)
        return _

    lax.fori_loop(0, NCHUNK // NBUF, step, None)

    for b in range(NBUF):
        pltpu.make_async_copy(bufs[b], out_hbm.at[pl.ds(0, CHUNK)],
                              wsems[b]).wait()


@jax.jit
def kernel(token, token_embedding_weight, positional_embedding):
    token_w = token.reshape(NW, NCHUNK, CHUNK).astype(jnp.int32)
    pos_ext = jnp.concatenate(
        [positional_embedding, positional_embedding[:POS_EXT - N_TOKEN]], 0)
    run = pl.kernel(
        _emb_kernel,
        out_type=jax.ShapeDtypeStruct((TOTAL, N_EMBED), jnp.float32),
        mesh=plsc.VectorSubcoreMesh(core_axis_name="c", subcore_axis_name="s"),
        scratch_types=(
            [pltpu.VMEM((NCHUNK, CHUNK), jnp.int32)]
            + [pltpu.VMEM((CHUNK, N_EMBED), jnp.float32)
               for _ in range(NBUF)]
            + [pltpu.SemaphoreType.DMA] * (3 * NBUF)
        ),
    )
    out = run(token_w, token_embedding_weight, pos_ext)
    return out.reshape(BATCH, N_TOKEN, N_EMBED)


# speedup vs baseline: 2.8484x; 2.8484x over previous
import jax
import jax.numpy as jnp
from jax import lax
from jax.experimental import pallas as pl
from jax.experimental.pallas import tpu as pltpu
from jax.experimental.pallas import tpu_sc as plsc

N_VOCAB = 100000
N_EMBED = 128
N_TOKEN = 200
BATCH = 4096

NC = 2
NS = 16
NW = NC * NS
LANES = 16

TOTAL = BATCH * N_TOKEN
PER_W = TOTAL // NW
CHUNK = 128
NCHUNK = PER_W // CHUNK
VEC_PER_ROW = N_EMBED // LANES


def _emb_kernel(token_hbm, table_hbm, pos_hbm, out_hbm,
                idx_v, pos_v, gbuf0, gbuf1, sbuf0, sbuf1,
                gsem0, gsem1, wsem0, wsem1):
    wid = lax.axis_index("s") * NC + lax.axis_index("c")
    base = wid * PER_W

    pltpu.sync_copy(token_hbm.at[wid], idx_v)
    pltpu.sync_copy(pos_hbm, pos_v)

    gbufs = (gbuf0, gbuf1)
    sbufs = (sbuf0, sbuf1)
    gsems = (gsem0, gsem1)
    wsems = (wsem0, wsem1)

    for b in range(2):
        pltpu.async_copy(table_hbm.at[idx_v.at[b]], gbufs[b], gsems[b])

    def step(k, _):
        for b in range(2):
            h = 2 * k + b
            pltpu.make_async_copy(table_hbm.at[idx_v.at[b]], gbufs[b],
                                  gsems[b]).wait()

            @pl.when(k >= 1)
            def _wait_store(b=b):
                pltpu.make_async_copy(
                    sbufs[b], out_hbm.at[pl.ds(0, CHUNK)], wsems[b]).wait()

            off = lax.rem(h * CHUNK, N_TOKEN)

            def row(j, b=b, off=off):
                p = off + j
                p = jnp.where(p >= N_TOKEN, p - N_TOKEN, p)
                for c in range(VEC_PER_ROW):
                    sl = pl.ds(c * LANES, LANES)
                    sbufs[b][j, sl] = gbufs[b][j, sl] + pos_v[p, sl]
            plsc.parallel_loop(0, CHUNK, 1, unroll=2, carry=None)(row)

            pltpu.async_copy(sbufs[b],
                             out_hbm.at[pl.ds(base + h * CHUNK, CHUNK)],
                             wsems[b])

            @pl.when(h + 2 < NCHUNK)
            def _next_gather(b=b, h=h):
                pltpu.async_copy(table_hbm.at[idx_v.at[h + 2]], gbufs[b],
                                 gsems[b])
        return _

    lax.fori_loop(0, NCHUNK // 2, step, None)

    for b in range(2):
        pltpu.make_async_copy(sbufs[b], out_hbm.at[pl.ds(0, CHUNK)],
                              wsems[b]).wait()


@jax.jit
def kernel(token, token_embedding_weight, positional_embedding):
    token_w = token.reshape(NW, NCHUNK, CHUNK).astype(jnp.int32)
    run = pl.kernel(
        _emb_kernel,
        out_type=jax.ShapeDtypeStruct((TOTAL, N_EMBED), jnp.float32),
        mesh=plsc.VectorSubcoreMesh(core_axis_name="c", subcore_axis_name="s"),
        scratch_types=(
            [pltpu.VMEM((NCHUNK, CHUNK), jnp.int32),
             pltpu.VMEM((N_TOKEN, N_EMBED), jnp.float32)]
            + [pltpu.VMEM((CHUNK, N_EMBED), jnp.float32)
               for _ in range(4)]
            + [pltpu.SemaphoreType.DMA] * 4
        ),
    )
    out = run(token_w, token_embedding_weight, positional_embedding)
    return out.reshape(BATCH, N_TOKEN, N_EMBED)


# speedup vs baseline: 2.8557x; 1.0026x over previous
import jax
import jax.numpy as jnp
from jax import lax
from jax.experimental import pallas as pl
from jax.experimental.pallas import tpu as pltpu
from jax.experimental.pallas import tpu_sc as plsc

N_VOCAB = 100000
N_EMBED = 128
N_TOKEN = 200
BATCH = 4096

NC = 2
NS = 16
NW = NC * NS
LANES = 16

TOTAL = BATCH * N_TOKEN
PER_W = TOTAL // NW
CHUNK = 128
NCHUNK = PER_W // CHUNK
VEC_PER_ROW = N_EMBED // LANES


def _emb_kernel(token_hbm, table_hbm, pos_hbm, out_hbm,
                idx_v, pos_v, gbuf0, gbuf1, sbuf0, sbuf1,
                gsem0, gsem1, wsem0, wsem1):
    wid = lax.axis_index("s") * NC + lax.axis_index("c")
    base = wid * PER_W

    pltpu.sync_copy(token_hbm.at[wid], idx_v)
    pltpu.sync_copy(pos_hbm, pos_v)

    gbufs = (gbuf0, gbuf1)
    sbufs = (sbuf0, sbuf1)
    gsems = (gsem0, gsem1)
    wsems = (wsem0, wsem1)

    for b in range(2):
        pltpu.async_copy(table_hbm.at[idx_v.at[b]], gbufs[b], gsems[b])

    def step(k, _):
        for b in range(2):
            h = 2 * k + b
            pltpu.make_async_copy(table_hbm.at[idx_v.at[b]], gbufs[b],
                                  gsems[b]).wait()

            @pl.when(k >= 1)
            def _wait_store(b=b):
                pltpu.make_async_copy(
                    sbufs[b], out_hbm.at[pl.ds(0, CHUNK)], wsems[b]).wait()

            off = lax.rem(h * CHUNK, N_TOKEN)

            def row(j, b=b, off=off):
                p = off + j
                p = jnp.where(p >= N_TOKEN, p - N_TOKEN, p)
                for c in range(VEC_PER_ROW):
                    sl = pl.ds(c * LANES, LANES)
                    sbufs[b][j, sl] = gbufs[b][j, sl] + pos_v[p, sl]
            plsc.parallel_loop(0, CHUNK, 1, unroll=4, carry=None)(row)

            pltpu.async_copy(sbufs[b],
                             out_hbm.at[pl.ds(base + h * CHUNK, CHUNK)],
                             wsems[b])

            @pl.when(h + 2 < NCHUNK)
            def _next_gather(b=b, h=h):
                pltpu.async_copy(table_hbm.at[idx_v.at[h + 2]], gbufs[b],
                                 gsems[b])
        return _

    lax.fori_loop(0, NCHUNK // 2, step, None)

    for b in range(2):
        pltpu.make_async_copy(sbufs[b], out_hbm.at[pl.ds(0, CHUNK)],
                              wsems[b]).wait()


@jax.jit
def kernel(token, token_embedding_weight, positional_embedding):
    token_w = token.reshape(NW, NCHUNK, CHUNK).astype(jnp.int32)
    run = pl.kernel(
        _emb_kernel,
        out_type=jax.ShapeDtypeStruct((TOTAL, N_EMBED), jnp.float32),
        mesh=plsc.VectorSubcoreMesh(core_axis_name="c", subcore_axis_name="s"),
        scratch_types=(
            [pltpu.VMEM((NCHUNK, CHUNK), jnp.int32),
             pltpu.VMEM((N_TOKEN, N_EMBED), jnp.float32)]
            + [pltpu.VMEM((CHUNK, N_EMBED), jnp.float32)
               for _ in range(4)]
            + [pltpu.SemaphoreType.DMA] * 4
        ),
    )
    out = run(token_w, token_embedding_weight, positional_embedding)
    return out.reshape(BATCH, N_TOKEN, N_EMBED)


# speedup vs baseline: 3.1433x; 1.1007x over previous
import jax
import jax.numpy as jnp
from jax import lax
from jax.experimental import pallas as pl
from jax.experimental.pallas import tpu as pltpu
from jax.experimental.pallas import tpu_sc as plsc

N_VOCAB = 100000
N_EMBED = 128
N_TOKEN = 200
BATCH = 4096

NC = 2
NS = 16
NW = NC * NS
LANES = 16

TOTAL = BATCH * N_TOKEN
B_PER_W = BATCH // NW
CHUNK = B_PER_W
NCHUNK = N_TOKEN
VEC_PER_ROW = N_EMBED // LANES


def _emb_kernel(token_hbm, table_hbm, pos_hbm, out_hbm,
                idx_v, pos_v, obase_v, oidx_v,
                gbuf0, gbuf1, sbuf0, sbuf1,
                gsem0, gsem1, wsem0, wsem1):
    wid = lax.axis_index("s") * NC + lax.axis_index("c")
    out_base = wid * (B_PER_W * N_TOKEN)

    pltpu.sync_copy(token_hbm.at[wid], idx_v)
    pltpu.sync_copy(pos_hbm, pos_v)

    lane = lax.iota(jnp.int32, LANES)
    for c in range(VEC_PER_ROW):
        obase_v[pl.ds(c * LANES, LANES)] = (
            out_base + (c * LANES) * N_TOKEN + lane * N_TOKEN)

    gbufs = (gbuf0, gbuf1)
    sbufs = (sbuf0, sbuf1)
    gsems = (gsem0, gsem1)
    wsems = (wsem0, wsem1)

    for b in range(2):
        pltpu.async_copy(table_hbm.at[idx_v.at[b]], gbufs[b], gsems[b])

    def step(k, _):
        for b in range(2):
            t = 2 * k + b
            pltpu.make_async_copy(table_hbm.at[idx_v.at[0]], gbufs[b],
                                  gsems[b]).wait()

            @pl.when(k >= 1)
            def _wait_store(b=b):
                pltpu.make_async_copy(
                    sbufs[b], out_hbm.at[oidx_v.at[b]], wsems[b]).wait()

            for c in range(VEC_PER_ROW):
                sl = pl.ds(c * LANES, LANES)
                oidx_v[b, sl] = obase_v[sl] + t

            pv = [pos_v[t, pl.ds(c * LANES, LANES)]
                  for c in range(VEC_PER_ROW)]

            def row(j, b=b, pv=pv):
                for c in range(VEC_PER_ROW):
                    sl = pl.ds(c * LANES, LANES)
                    sbufs[b][j, sl] = gbufs[b][j, sl] + pv[c]
            plsc.parallel_loop(0, CHUNK, 1, unroll=4, carry=None)(row)

            pltpu.async_copy(sbufs[b], out_hbm.at[oidx_v.at[b]], wsems[b])

            @pl.when(t + 2 < NCHUNK)
            def _next_gather(b=b, t=t):
                pltpu.async_copy(table_hbm.at[idx_v.at[t + 2]], gbufs[b],
                                 gsems[b])
        return _

    lax.fori_loop(0, NCHUNK // 2, step, None)

    for b in range(2):
        pltpu.make_async_copy(sbufs[b], out_hbm.at[oidx_v.at[b]],
                              wsems[b]).wait()


@jax.jit
def kernel(token, token_embedding_weight, positional_embedding):
    token_t = (token.reshape(NW, B_PER_W, N_TOKEN)
               .transpose(0, 2, 1).astype(jnp.int32))
    run = pl.kernel(
        _emb_kernel,
        out_type=jax.ShapeDtypeStruct((TOTAL, N_EMBED), jnp.float32),
        mesh=plsc.VectorSubcoreMesh(core_axis_name="c", subcore_axis_name="s"),
        scratch_types=(
            [pltpu.VMEM((NCHUNK, CHUNK), jnp.int32),
             pltpu.VMEM((N_TOKEN, N_EMBED), jnp.float32),
             pltpu.VMEM((CHUNK,), jnp.int32),
             pltpu.VMEM((2, CHUNK), jnp.int32)]
            + [pltpu.VMEM((CHUNK, N_EMBED), jnp.float32)
               for _ in range(4)]
            + [pltpu.SemaphoreType.DMA] * 4
        ),
    )
    out = run(token_t, token_embedding_weight, positional_embedding)
    return out.reshape(BATCH, N_TOKEN, N_EMBED)
